# Initial kernel scaffold; baseline (speedup 1.0000x reference)
import jax
import jax.numpy as jnp
from jax.experimental import pallas as pl


def kernel(x, W1_0, b1_0, W2_0, b2_0, W1_1, b1_1, W2_1, b2_1, W1_2, b1_2, W2_2, b2_2, Wl, bl, edge_index, batch):
    raise NotImplementedError("write your pallas kernel here")


# speedup vs baseline: 8.9180x; 8.9180x over previous
import jax
import jax.numpy as jnp
from jax import lax
from jax.experimental import pallas as pl
from jax.experimental.pallas import tpu as pltpu
from jax.experimental.pallas import tpu_sc as plsc

_N = 10000
_E = 320000
_D = 128
_H = 64
_G = 64
_C = 2

_NCORES = 2
_NSUB = 16
_NTILES = _NCORES * _NSUB
_EPT = _E // _NTILES
_CH = 80
_NCH = _EPT // _CH
_NPAD = 10240
_RPT = _NPAD // _NSUB
_RCH = 128
_NRC = _RPT // _RCH

_BLK = 2000
_NBLK = _N // _BLK


def _sc_agg_body(p_hbm, src_hbm, dst_hbm, q_hbm,
                 src_v, dst_v, rows_v, buf_v, acc_sh, sem):
    c = lax.axis_index("c")
    s = lax.axis_index("s")
    wid = c * _NSUB + s

    def _zfill(k, carry):
        buf_v[k // 4, pl.ds((k % 4) * 16, 16)] = jnp.zeros((16,), jnp.float32)
        return carry
    lax.fori_loop(0, _RCH * 4, _zfill, 0)

    def _zcopy(r, carry):
        pltpu.sync_copy(buf_v, acc_sh.at[pl.ds(s * _RPT + r * _RCH, _RCH)])
        return carry
    lax.fori_loop(0, _NRC, _zcopy, 0)
    plsc.subcore_barrier()

    pltpu.sync_copy(src_hbm.at[wid], src_v)
    pltpu.sync_copy(dst_hbm.at[wid], dst_v)

    def _edges(j, carry):
        pltpu.async_copy(p_hbm.at[src_v.at[j]], rows_v, sem).wait()
        pltpu.sync_copy(rows_v, acc_sh.at[dst_v.at[j]], add=True)
        return carry
    lax.fori_loop(0, _NCH, _edges, 0)
    plsc.subcore_barrier()

    def _wb(r, carry):
        r0 = s * _RPT + r * _RCH
        pltpu.sync_copy(acc_sh.at[pl.ds(r0, _RCH)], buf_v)
        pltpu.sync_copy(buf_v, q_hbm.at[c, pl.ds(r0, _RCH)])
        return carry
    lax.fori_loop(0, _NRC, _wb, 0)


_sc_agg = pl.kernel(
    _sc_agg_body,
    out_type=jax.ShapeDtypeStruct((_NCORES, _NPAD, _H), jnp.float32),
    mesh=plsc.VectorSubcoreMesh(core_axis_name="c", subcore_axis_name="s"),
    scratch_types=[
        pltpu.VMEM((_NCH, _CH), jnp.int32),
        pltpu.VMEM((_NCH, _CH), jnp.int32),
        pltpu.VMEM((_CH, _H), jnp.float32),
        pltpu.VMEM((_RCH, _H), jnp.float32),
        pltpu.VMEM_SHARED((_NPAD, _H), jnp.float32),
        pltpu.SemaphoreType.DMA,
    ],
    compiler_params=pltpu.CompilerParams(use_tc_tiling_on_sc=False),
)


def _k0_body(x_ref, w_ref, o_ref):
    o_ref[...] = jnp.dot(x_ref[...], w_ref[...],
                         preferred_element_type=jnp.float32)


_k0 = pl.pallas_call(
    _k0_body,
    grid=(_NBLK,),
    in_specs=[
        pl.BlockSpec((_BLK, _D), lambda i: (i, 0)),
        pl.BlockSpec((_D, _H), lambda i: (0, 0)),
    ],
    out_specs=pl.BlockSpec((_BLK, _H), lambda i: (i, 0)),
    out_shape=jax.ShapeDtypeStruct((_N, _H), jnp.float32),
)


def _kmid_body(p_ref, q0_ref, q1_ref, b1_ref, w2_ref, b2_ref, w1n_ref, o_ref):
    z = jnp.maximum(p_ref[...] + q0_ref[0] + q1_ref[0] + b1_ref[...], 0.0)
    t = jnp.dot(z, w2_ref[...], preferred_element_type=jnp.float32) + b2_ref[...]
    t = jnp.maximum(t, 0.0)
    o_ref[...] = jnp.dot(t, w1n_ref[...], preferred_element_type=jnp.float32)


_kmid = pl.pallas_call(
    _kmid_body,
    grid=(_NBLK,),
    in_specs=[
        pl.BlockSpec((_BLK, _H), lambda i: (i, 0)),
        pl.BlockSpec((1, _BLK, _H), lambda i: (0, i, 0)),
        pl.BlockSpec((1, _BLK, _H), lambda i: (1, i, 0)),
        pl.BlockSpec((1, _H), lambda i: (0, 0)),
        pl.BlockSpec((_H, _H), lambda i: (0, 0)),
        pl.BlockSpec((1, _H), lambda i: (0, 0)),
        pl.BlockSpec((_H, _H), lambda i: (0, 0)),
    ],
    out_specs=pl.BlockSpec((_BLK, _H), lambda i: (i, 0)),
    out_shape=jax.ShapeDtypeStruct((_N, _H), jnp.float32),
)


def _kfin_body(p_ref, q0_ref, q1_ref, b1_ref, w2_ref, b2_ref, bt_ref,
               wl_ref, bl_ref, out_ref, g_ref, sums_ref, cnt_ref):
    i = pl.program_id(0)
    z = jnp.maximum(p_ref[...] + q0_ref[0] + q1_ref[0] + b1_ref[...], 0.0)
    h = jnp.dot(z, w2_ref[...], preferred_element_type=jnp.float32) + b2_ref[...]
    h = jnp.maximum(h, 0.0)

    bt = bt_ref[0, 0, :]
    onehot = (bt[:, None] == lax.broadcasted_iota(jnp.int32, (1, _G), 1)
              ).astype(jnp.float32)
    part = lax.dot_general(onehot, h, (((0,), (0,)), ((), ())),
                           preferred_element_type=jnp.float32)
    pcnt = lax.dot_general(onehot, jnp.ones((_BLK, 1), jnp.float32),
                           (((0,), (0,)), ((), ())),
                           preferred_element_type=jnp.float32)

    @pl.when(i == 0)
    def _():
        sums_ref[...] = part
        cnt_ref[...] = pcnt

    @pl.when(i > 0)
    def _():
        sums_ref[...] += part
        cnt_ref[...] += pcnt

    @pl.when(i == pl.num_programs(0) - 1)
    def _():
        g = sums_ref[...] / jnp.maximum(cnt_ref[...], 1.0)
        g_ref[...] = g
        out_ref[...] = jnp.dot(g, wl_ref[...],
                               preferred_element_type=jnp.float32) + bl_ref[...]


_kfin = pl.pallas_call(
    _kfin_body,
    grid=(_NBLK,),
    in_specs=[
        pl.BlockSpec((_BLK, _H), lambda i: (i, 0)),
        pl.BlockSpec((1, _BLK, _H), lambda i: (0, i, 0)),
        pl.BlockSpec((1, _BLK, _H), lambda i: (1, i, 0)),
        pl.BlockSpec((1, _H), lambda i: (0, 0)),
        pl.BlockSpec((_H, _H), lambda i: (0, 0)),
        pl.BlockSpec((1, _H), lambda i: (0, 0)),
        pl.BlockSpec((1, 1, _BLK), lambda i: (i, 0, 0)),
        pl.BlockSpec((_H, _C), lambda i: (0, 0)),
        pl.BlockSpec((1, _C), lambda i: (0, 0)),
    ],
    out_specs=[
        pl.BlockSpec((_G, _C), lambda i: (0, 0)),
        pl.BlockSpec((_G, _H), lambda i: (0, 0)),
    ],
    out_shape=[
        jax.ShapeDtypeStruct((_G, _C), jnp.float32),
        jax.ShapeDtypeStruct((_G, _H), jnp.float32),
    ],
    scratch_shapes=[
        pltpu.VMEM((_G, _H), jnp.float32),
        pltpu.VMEM((_G, 1), jnp.float32),
    ],
)


def kernel(x, W1_0, b1_0, W2_0, b2_0, W1_1, b1_1, W2_1, b2_1,
           W1_2, b1_2, W2_2, b2_2, Wl, bl, edge_index, batch):
    src = edge_index[0].reshape(_NTILES, _NCH, _CH)
    dst = edge_index[1].reshape(_NTILES, _NCH, _CH)
    bt3 = batch.reshape(_NBLK, 1, _BLK)
    b1_0r, b2_0r = b1_0.reshape(1, _H), b2_0.reshape(1, _H)
    b1_1r, b2_1r = b1_1.reshape(1, _H), b2_1.reshape(1, _H)
    b1_2r, b2_2r = b1_2.reshape(1, _H), b2_2.reshape(1, _H)
    bl_r = bl.reshape(1, _C)

    p = _k0(x, W1_0)
    q = _sc_agg(p, src, dst)
    p = _kmid(p, q, q, b1_0r, W2_0, b2_0r, W1_1)
    q = _sc_agg(p, src, dst)
    p = _kmid(p, q, q, b1_1r, W2_1, b2_1r, W1_2)
    q = _sc_agg(p, src, dst)
    out, g = _kfin(p, q, q, b1_2r, W2_2, b2_2r, bt3, Wl, bl_r)
    return (out, g)


# speedup vs baseline: 13.6925x; 1.5354x over previous
import jax
import jax.numpy as jnp
from jax import lax
from jax.experimental import pallas as pl
from jax.experimental.pallas import tpu as pltpu
from jax.experimental.pallas import tpu_sc as plsc

_N = 10000
_E = 320000
_D = 128
_H = 64
_G = 64
_C = 2

_NCORES = 2
_NSUB = 16
_NTILES = _NCORES * _NSUB
_EPT = _E // _NTILES
_CH = 80
_NCH = _EPT // _CH
_NPAD = 10240
_RPT = _NPAD // _NSUB
_RCH = 128
_NRC = _RPT // _RCH

_BLK = 2000
_NBLK = _N // _BLK


def _sc_agg_body(p_hbm, src_hbm, dst_hbm, q_hbm,
                 src_v, dst_v, rows_v, buf_v, acc_sh, sem):
    c = lax.axis_index("c")
    s = lax.axis_index("s")
    wid = c * _NSUB + s

    def _zfill(k, carry):
        buf_v[k // 4, pl.ds((k % 4) * 16, 16)] = jnp.zeros((16,), jnp.float32)
        return carry
    lax.fori_loop(0, _RCH * 4, _zfill, 0)

    def _zcopy(r, carry):
        pltpu.sync_copy(buf_v, acc_sh.at[pl.ds(s * _RPT + r * _RCH, _RCH)])
        return carry
    lax.fori_loop(0, _NRC, _zcopy, 0)
    plsc.subcore_barrier()

    pltpu.sync_copy(src_hbm.at[wid], src_v)
    pltpu.sync_copy(dst_hbm.at[wid], dst_v)

    pltpu.async_copy(p_hbm.at[src_v.at[0]], rows_v.at[0], sem.at[0])

    def _edges(j, carry):
        b = lax.rem(j, 2)
        nb = lax.rem(j + 1, 2)

        @pl.when(j + 1 < _NCH)
        def _():
            pltpu.async_copy(p_hbm.at[src_v.at[j + 1]], rows_v.at[nb],
                             sem.at[nb])

        pltpu.make_async_copy(p_hbm.at[src_v.at[j]], rows_v.at[b],
                              sem.at[b]).wait()
        pltpu.sync_copy(rows_v.at[b], acc_sh.at[dst_v.at[j]], add=True)
        return carry
    lax.fori_loop(0, _NCH, _edges, 0)
    plsc.subcore_barrier()

    def _wb(r, carry):
        r0 = s * _RPT + r * _RCH
        pltpu.sync_copy(acc_sh.at[pl.ds(r0, _RCH)], buf_v)
        pltpu.sync_copy(buf_v, q_hbm.at[c, pl.ds(r0, _RCH)])
        return carry
    lax.fori_loop(0, _NRC, _wb, 0)


_sc_agg = pl.kernel(
    _sc_agg_body,
    out_type=jax.ShapeDtypeStruct((_NCORES, _NPAD, _H), jnp.float32),
    mesh=plsc.VectorSubcoreMesh(core_axis_name="c", subcore_axis_name="s"),
    scratch_types=[
        pltpu.VMEM((_NCH, _CH), jnp.int32),
        pltpu.VMEM((_NCH, _CH), jnp.int32),
        pltpu.VMEM((2, _CH, _H), jnp.float32),
        pltpu.VMEM((_RCH, _H), jnp.float32),
        pltpu.VMEM_SHARED((_NPAD, _H), jnp.float32),
        pltpu.SemaphoreType.DMA((2,)),
    ],
    compiler_params=pltpu.CompilerParams(use_tc_tiling_on_sc=False),
)


def _k0_body(x_ref, w_ref, o_ref):
    o_ref[...] = jnp.dot(x_ref[...], w_ref[...],
                         preferred_element_type=jnp.float32)


_k0 = pl.pallas_call(
    _k0_body,
    grid=(_NBLK,),
    in_specs=[
        pl.BlockSpec((_BLK, _D), lambda i: (i, 0)),
        pl.BlockSpec((_D, _H), lambda i: (0, 0)),
    ],
    out_specs=pl.BlockSpec((_BLK, _H), lambda i: (i, 0)),
    out_shape=jax.ShapeDtypeStruct((_N, _H), jnp.float32),
)


def _kmid_body(p_ref, q0_ref, q1_ref, b1_ref, w2_ref, b2_ref, w1n_ref, o_ref):
    z = jnp.maximum(p_ref[...] + q0_ref[0] + q1_ref[0] + b1_ref[...], 0.0)
    t = jnp.dot(z, w2_ref[...], preferred_element_type=jnp.float32) + b2_ref[...]
    t = jnp.maximum(t, 0.0)
    o_ref[...] = jnp.dot(t, w1n_ref[...], preferred_element_type=jnp.float32)


_kmid = pl.pallas_call(
    _kmid_body,
    grid=(_NBLK,),
    in_specs=[
        pl.BlockSpec((_BLK, _H), lambda i: (i, 0)),
        pl.BlockSpec((1, _BLK, _H), lambda i: (0, i, 0)),
        pl.BlockSpec((1, _BLK, _H), lambda i: (1, i, 0)),
        pl.BlockSpec((1, _H), lambda i: (0, 0)),
        pl.BlockSpec((_H, _H), lambda i: (0, 0)),
        pl.BlockSpec((1, _H), lambda i: (0, 0)),
        pl.BlockSpec((_H, _H), lambda i: (0, 0)),
    ],
    out_specs=pl.BlockSpec((_BLK, _H), lambda i: (i, 0)),
    out_shape=jax.ShapeDtypeStruct((_N, _H), jnp.float32),
)


def _kfin_body(p_ref, q0_ref, q1_ref, b1_ref, w2_ref, b2_ref, bt_ref,
               wl_ref, bl_ref, out_ref, g_ref, sums_ref, cnt_ref):
    i = pl.program_id(0)
    z = jnp.maximum(p_ref[...] + q0_ref[0] + q1_ref[0] + b1_ref[...], 0.0)
    h = jnp.dot(z, w2_ref[...], preferred_element_type=jnp.float32) + b2_ref[...]
    h = jnp.maximum(h, 0.0)

    bt = bt_ref[0, 0, :]
    onehot = (bt[:, None] == lax.broadcasted_iota(jnp.int32, (1, _G), 1)
              ).astype(jnp.float32)
    part = lax.dot_general(onehot, h, (((0,), (0,)), ((), ())),
                           preferred_element_type=jnp.float32)
    pcnt = lax.dot_general(onehot, jnp.ones((_BLK, 1), jnp.float32),
                           (((0,), (0,)), ((), ())),
                           preferred_element_type=jnp.float32)

    @pl.when(i == 0)
    def _():
        sums_ref[...] = part
        cnt_ref[...] = pcnt

    @pl.when(i > 0)
    def _():
        sums_ref[...] += part
        cnt_ref[...] += pcnt

    @pl.when(i == pl.num_programs(0) - 1)
    def _():
        g = sums_ref[...] / jnp.maximum(cnt_ref[...], 1.0)
        g_ref[...] = g
        out_ref[...] = jnp.dot(g, wl_ref[...],
                               preferred_element_type=jnp.float32) + bl_ref[...]


_kfin = pl.pallas_call(
    _kfin_body,
    grid=(_NBLK,),
    in_specs=[
        pl.BlockSpec((_BLK, _H), lambda i: (i, 0)),
        pl.BlockSpec((1, _BLK, _H), lambda i: (0, i, 0)),
        pl.BlockSpec((1, _BLK, _H), lambda i: (1, i, 0)),
        pl.BlockSpec((1, _H), lambda i: (0, 0)),
        pl.BlockSpec((_H, _H), lambda i: (0, 0)),
        pl.BlockSpec((1, _H), lambda i: (0, 0)),
        pl.BlockSpec((1, 1, _BLK), lambda i: (i, 0, 0)),
        pl.BlockSpec((_H, _C), lambda i: (0, 0)),
        pl.BlockSpec((1, _C), lambda i: (0, 0)),
    ],
    out_specs=[
        pl.BlockSpec((_G, _C), lambda i: (0, 0)),
        pl.BlockSpec((_G, _H), lambda i: (0, 0)),
    ],
    out_shape=[
        jax.ShapeDtypeStruct((_G, _C), jnp.float32),
        jax.ShapeDtypeStruct((_G, _H), jnp.float32),
    ],
    scratch_shapes=[
        pltpu.VMEM((_G, _H), jnp.float32),
        pltpu.VMEM((_G, 1), jnp.float32),
    ],
)


def kernel(x, W1_0, b1_0, W2_0, b2_0, W1_1, b1_1, W2_1, b2_1,
           W1_2, b1_2, W2_2, b2_2, Wl, bl, edge_index, batch):
    src = edge_index[0].reshape(_NTILES, _NCH, _CH)
    dst = edge_index[1].reshape(_NTILES, _NCH, _CH)
    bt3 = batch.reshape(_NBLK, 1, _BLK)
    b1_0r, b2_0r = b1_0.reshape(1, _H), b2_0.reshape(1, _H)
    b1_1r, b2_1r = b1_1.reshape(1, _H), b2_1.reshape(1, _H)
    b1_2r, b2_2r = b1_2.reshape(1, _H), b2_2.reshape(1, _H)
    bl_r = bl.reshape(1, _C)

    p = _k0(x, W1_0)
    q = _sc_agg(p, src, dst)
    p = _kmid(p, q, q, b1_0r, W2_0, b2_0r, W1_1)
    q = _sc_agg(p, src, dst)
    p = _kmid(p, q, q, b1_1r, W2_1, b2_1r, W1_2)
    q = _sc_agg(p, src, dst)
    out, g = _kfin(p, q, q, b1_2r, W2_2, b2_2r, bt3, Wl, bl_r)
    return (out, g)
